# baseline (speedup 1.0000x reference)
import jax
import jax.numpy as jnp
from jax.experimental import pallas as pl
from jax.experimental.pallas import tpu as pltpu

_N = 32768
_B = 16
_ESM = 1280
_P = 128
_H = 64
_BLK = 2048
_NEG = -1e30


def _fused(x_ref, ids_ref, w1_ref, b1_ref, g_ref, be_ref, ws1t_ref, bs1t_ref,
           ws2t_ref, bs2_ref, out_ref, m_ref, den_ref, cnt_ref, zsum_ref,
           attn_ref):
    i = pl.program_id(0)
    nsteps = pl.num_programs(0)

    @pl.when(i == 0)
    def _init():
        m_ref[...] = jnp.full_like(m_ref, _NEG)
        den_ref[...] = jnp.zeros_like(den_ref)
        cnt_ref[...] = jnp.zeros_like(cnt_ref)
        zsum_ref[...] = jnp.zeros_like(zsum_ref)
        attn_ref[...] = jnp.zeros_like(attn_ref)

    x = x_ref[...]
    h = jax.lax.dot_general(x, w1_ref[...],
                            (((1,), (0,)), ((), ())),
                            preferred_element_type=jnp.float32) + b1_ref[...]

    ones_col = jnp.ones((_P, 1), dtype=jnp.float32)
    mu = jax.lax.dot_general(h, ones_col, (((1,), (0,)), ((), ())),
                             preferred_element_type=jnp.float32) * (1.0 / _P)
    d = h - mu
    var = jax.lax.dot_general(d * d, ones_col, (((1,), (0,)), ((), ())),
                              preferred_element_type=jnp.float32) * (1.0 / _P)
    r = jax.lax.rsqrt(var + 1e-5)
    hn = d * r * g_ref[...] + be_ref[...]
    z = hn * jax.nn.sigmoid(hn)

    sT = jax.lax.dot_general(ws1t_ref[...], z, (((1,), (1,)), ((), ())),
                             preferred_element_type=jnp.float32) + bs1t_ref[...]
    sT = sT * jax.nn.sigmoid(sT)
    logits = jax.lax.dot_general(ws2t_ref[...], sT, (((1,), (0,)), ((), ())),
                                 preferred_element_type=jnp.float32) + bs2_ref[...]

    ids = ids_ref[0]
    seg = jax.lax.broadcasted_iota(jnp.int32, (_B, 1), 0)
    onehot = ids == seg
    oh_f = onehot.astype(jnp.float32)

    bcnt = jnp.sum(oh_f, axis=1, keepdims=True)
    bzsum = jax.lax.dot_general(oh_f, z, (((1,), (0,)), ((), ())),
                                preferred_element_type=jnp.float32)

    masked = jnp.where(onehot, logits, _NEG)
    bmax = jnp.max(masked, axis=1, keepdims=True)
    e = jnp.exp(jnp.where(onehot, logits - bmax, _NEG))
    bden = jnp.sum(e, axis=1, keepdims=True)
    battn = jax.lax.dot_general(e, z, (((1,), (0,)), ((), ())),
                                preferred_element_type=jnp.float32)

    m_old = m_ref[...]
    m_new = jnp.maximum(m_old, bmax)
    alpha = jnp.exp(m_old - m_new)
    gamma = jnp.exp(bmax - m_new)
    m_ref[...] = m_new
    den_ref[...] = den_ref[...] * alpha + bden * gamma
    cnt_ref[...] = cnt_ref[...] + bcnt
    attn_ref[...] = attn_ref[...] * alpha + battn * gamma
    zsum_ref[...] = zsum_ref[...] + bzsum

    @pl.when(i == nsteps - 1)
    def _fin():
        zmean = zsum_ref[...] / jnp.maximum(cnt_ref[...], 1.0)
        zattn = attn_ref[...] / (den_ref[...] + 1e-16)
        out_ref[:, :_P] = zmean
        out_ref[:, _P:] = zattn


def kernel(x_esm, batch, W1, b1, gamma, beta, Ws1, bs1, Ws2, bs2):
    ids = batch.astype(jnp.int32).reshape(_N // _BLK, 1, _BLK)
    out = pl.pallas_call(
        _fused,
        grid=(_N // _BLK,),
        in_specs=[
            pl.BlockSpec((_BLK, _ESM), lambda i: (i, 0)),
            pl.BlockSpec((1, 1, _BLK), lambda i: (i, 0, 0)),
            pl.BlockSpec((_ESM, _P), lambda i: (0, 0)),
            pl.BlockSpec((1, _P), lambda i: (0, 0)),
            pl.BlockSpec((1, _P), lambda i: (0, 0)),
            pl.BlockSpec((1, _P), lambda i: (0, 0)),
            pl.BlockSpec((_H, _P), lambda i: (0, 0)),
            pl.BlockSpec((_H, 1), lambda i: (0, 0)),
            pl.BlockSpec((1, _H), lambda i: (0, 0)),
            pl.BlockSpec((1, 1), lambda i: (0, 0)),
        ],
        out_specs=pl.BlockSpec((_B, 2 * _P), lambda i: (0, 0)),
        out_shape=jax.ShapeDtypeStruct((_B, 2 * _P), jnp.float32),
        scratch_shapes=[
            pltpu.VMEM((_B, 1), jnp.float32),
            pltpu.VMEM((_B, 1), jnp.float32),
            pltpu.VMEM((_B, 1), jnp.float32),
            pltpu.VMEM((_B, _P), jnp.float32),
            pltpu.VMEM((_B, _P), jnp.float32),
        ],
        compiler_params=pltpu.CompilerParams(
            dimension_semantics=("arbitrary",)),
    )(x_esm, ids, W1, b1.reshape(1, _P), gamma.reshape(1, _P),
      beta.reshape(1, _P), Ws1.T, bs1.reshape(_H, 1), Ws2.T,
      bs2.reshape(1, 1))
    return out


# speedup vs baseline: 1.0476x; 1.0476x over previous
import jax
import jax.numpy as jnp
from jax.experimental import pallas as pl
from jax.experimental.pallas import tpu as pltpu

_N = 32768
_B = 16
_ESM = 1280
_P = 128
_H = 64
_BLK = 4096
_NEG = -1e30


def _fused(x_ref, ids_ref, w1_ref, b1_ref, g_ref, be_ref, ws1t_ref, bs1t_ref,
           ws2t_ref, bs2_ref, out_ref, m_ref, den_ref, cnt_ref, zsum_ref,
           attn_ref):
    i = pl.program_id(0)
    nsteps = pl.num_programs(0)

    @pl.when(i == 0)
    def _init():
        m_ref[...] = jnp.full_like(m_ref, _NEG)
        den_ref[...] = jnp.zeros_like(den_ref)
        cnt_ref[...] = jnp.zeros_like(cnt_ref)
        zsum_ref[...] = jnp.zeros_like(zsum_ref)
        attn_ref[...] = jnp.zeros_like(attn_ref)

    x = x_ref[...]
    h = jax.lax.dot_general(x, w1_ref[...],
                            (((1,), (0,)), ((), ())),
                            preferred_element_type=jnp.float32) + b1_ref[...]

    ones_col = jnp.ones((_P, 1), dtype=jnp.float32)
    mu = jax.lax.dot_general(h, ones_col, (((1,), (0,)), ((), ())),
                             preferred_element_type=jnp.float32) * (1.0 / _P)
    d = h - mu
    var = jax.lax.dot_general(d * d, ones_col, (((1,), (0,)), ((), ())),
                              preferred_element_type=jnp.float32) * (1.0 / _P)
    r = jax.lax.rsqrt(var + 1e-5)
    hn = d * r * g_ref[...] + be_ref[...]
    z = hn * jax.nn.sigmoid(hn)

    sT = jax.lax.dot_general(ws1t_ref[...], z, (((1,), (1,)), ((), ())),
                             preferred_element_type=jnp.float32) + bs1t_ref[...]
    sT = sT * jax.nn.sigmoid(sT)
    logits = jax.lax.dot_general(ws2t_ref[...], sT, (((1,), (0,)), ((), ())),
                                 preferred_element_type=jnp.float32) + bs2_ref[...]

    ids = ids_ref[0]
    seg = jax.lax.broadcasted_iota(jnp.int32, (_B, 1), 0)
    onehot = ids == seg
    oh_f = onehot.astype(jnp.float32)

    bcnt = jnp.sum(oh_f, axis=1, keepdims=True)
    bzsum = jax.lax.dot_general(oh_f, z, (((1,), (0,)), ((), ())),
                                preferred_element_type=jnp.float32)

    masked = jnp.where(onehot, logits, _NEG)
    bmax = jnp.max(masked, axis=1, keepdims=True)
    e = jnp.exp(jnp.where(onehot, logits - bmax, _NEG))
    bden = jnp.sum(e, axis=1, keepdims=True)
    battn = jax.lax.dot_general(e, z, (((1,), (0,)), ((), ())),
                                preferred_element_type=jnp.float32)

    m_old = m_ref[...]
    m_new = jnp.maximum(m_old, bmax)
    alpha = jnp.exp(m_old - m_new)
    gamma = jnp.exp(bmax - m_new)
    m_ref[...] = m_new
    den_ref[...] = den_ref[...] * alpha + bden * gamma
    cnt_ref[...] = cnt_ref[...] + bcnt
    attn_ref[...] = attn_ref[...] * alpha + battn * gamma
    zsum_ref[...] = zsum_ref[...] + bzsum

    @pl.when(i == nsteps - 1)
    def _fin():
        zmean = zsum_ref[...] / jnp.maximum(cnt_ref[...], 1.0)
        zattn = attn_ref[...] / (den_ref[...] + 1e-16)
        out_ref[:, :_P] = zmean
        out_ref[:, _P:] = zattn


def kernel(x_esm, batch, W1, b1, gamma, beta, Ws1, bs1, Ws2, bs2):
    ids = batch.astype(jnp.int32).reshape(_N // _BLK, 1, _BLK)
    out = pl.pallas_call(
        _fused,
        grid=(_N // _BLK,),
        in_specs=[
            pl.BlockSpec((_BLK, _ESM), lambda i: (i, 0)),
            pl.BlockSpec((1, 1, _BLK), lambda i: (i, 0, 0)),
            pl.BlockSpec((_ESM, _P), lambda i: (0, 0)),
            pl.BlockSpec((1, _P), lambda i: (0, 0)),
            pl.BlockSpec((1, _P), lambda i: (0, 0)),
            pl.BlockSpec((1, _P), lambda i: (0, 0)),
            pl.BlockSpec((_H, _P), lambda i: (0, 0)),
            pl.BlockSpec((_H, 1), lambda i: (0, 0)),
            pl.BlockSpec((1, _H), lambda i: (0, 0)),
            pl.BlockSpec((1, 1), lambda i: (0, 0)),
        ],
        out_specs=pl.BlockSpec((_B, 2 * _P), lambda i: (0, 0)),
        out_shape=jax.ShapeDtypeStruct((_B, 2 * _P), jnp.float32),
        scratch_shapes=[
            pltpu.VMEM((_B, 1), jnp.float32),
            pltpu.VMEM((_B, 1), jnp.float32),
            pltpu.VMEM((_B, 1), jnp.float32),
            pltpu.VMEM((_B, _P), jnp.float32),
            pltpu.VMEM((_B, _P), jnp.float32),
        ],
        compiler_params=pltpu.CompilerParams(
            dimension_semantics=("arbitrary",)),
    )(x_esm, ids, W1, b1.reshape(1, _P), gamma.reshape(1, _P),
      beta.reshape(1, _P), Ws1.T, bs1.reshape(_H, 1), Ws2.T,
      bs2.reshape(1, 1))
    return out


# speedup vs baseline: 1.0602x; 1.0121x over previous
import jax
import jax.numpy as jnp
from jax.experimental import pallas as pl
from jax.experimental.pallas import tpu as pltpu

_N = 32768
_B = 16
_ESM = 1280
_P = 128
_H = 64
_BLK = 4096
_NEG = -1e30


def _fused(x_ref, ids_ref, w1_ref, b1_ref, g_ref, be_ref, ws1t_ref, bs1t_ref,
           ws2t_ref, bs2_ref, out_ref, m_ref, den_ref, cnt_ref, zsum_ref,
           attn_ref):
    i = pl.program_id(0)
    nsteps = pl.num_programs(0)

    @pl.when(i == 0)
    def _init():
        m_ref[...] = jnp.full_like(m_ref, _NEG)
        den_ref[...] = jnp.zeros_like(den_ref)
        cnt_ref[...] = jnp.zeros_like(cnt_ref)
        zsum_ref[...] = jnp.zeros_like(zsum_ref)
        attn_ref[...] = jnp.zeros_like(attn_ref)

    x = x_ref[...]
    h = jax.lax.dot_general(x, w1_ref[...],
                            (((1,), (0,)), ((), ())),
                            preferred_element_type=jnp.float32) + b1_ref[...]

    ones_col = jnp.ones((_P, 1), dtype=jnp.float32)
    mu = jax.lax.dot_general(h, ones_col, (((1,), (0,)), ((), ())),
                             preferred_element_type=jnp.float32) * (1.0 / _P)
    d = h - mu
    var = jax.lax.dot_general(d * d, ones_col, (((1,), (0,)), ((), ())),
                              preferred_element_type=jnp.float32) * (1.0 / _P)
    r = jax.lax.rsqrt(var + 1e-5)
    hn = d * r * g_ref[...] + be_ref[...]
    z = hn * jax.nn.sigmoid(hn)

    sT = jax.lax.dot_general(ws1t_ref[...], z, (((1,), (1,)), ((), ())),
                             preferred_element_type=jnp.float32) + bs1t_ref[...]
    sT = sT * jax.nn.sigmoid(sT)
    logits = jax.lax.dot_general(ws2t_ref[...], sT, (((1,), (0,)), ((), ())),
                                 preferred_element_type=jnp.float32) + bs2_ref[...]

    ids = ids_ref[0]
    seg = jax.lax.broadcasted_iota(jnp.int32, (_B, 1), 0)
    onehot = ids == seg
    oh_f = onehot.astype(jnp.float32)

    bcnt = jnp.sum(oh_f, axis=1, keepdims=True)
    masked = jnp.where(onehot, logits, _NEG)
    bmax = jnp.max(masked, axis=1, keepdims=True)
    e = jnp.exp(jnp.where(onehot, logits - bmax, _NEG))
    bden = jnp.sum(e, axis=1, keepdims=True)
    ohe = jnp.concatenate([oh_f, e], axis=0)
    both = jax.lax.dot_general(ohe, z, (((1,), (0,)), ((), ())),
                               preferred_element_type=jnp.float32)
    bzsum = both[:_B]
    battn = both[_B:]

    m_old = m_ref[...]
    m_new = jnp.maximum(m_old, bmax)
    alpha = jnp.exp(m_old - m_new)
    gamma = jnp.exp(bmax - m_new)
    m_ref[...] = m_new
    den_ref[...] = den_ref[...] * alpha + bden * gamma
    cnt_ref[...] = cnt_ref[...] + bcnt
    attn_ref[...] = attn_ref[...] * alpha + battn * gamma
    zsum_ref[...] = zsum_ref[...] + bzsum

    @pl.when(i == nsteps - 1)
    def _fin():
        zmean = zsum_ref[...] / jnp.maximum(cnt_ref[...], 1.0)
        zattn = attn_ref[...] / (den_ref[...] + 1e-16)
        out_ref[:, :_P] = zmean
        out_ref[:, _P:] = zattn


def kernel(x_esm, batch, W1, b1, gamma, beta, Ws1, bs1, Ws2, bs2):
    ids = batch.astype(jnp.int32).reshape(_N // _BLK, 1, _BLK)
    out = pl.pallas_call(
        _fused,
        grid=(_N // _BLK,),
        in_specs=[
            pl.BlockSpec((_BLK, _ESM), lambda i: (i, 0)),
            pl.BlockSpec((1, 1, _BLK), lambda i: (i, 0, 0)),
            pl.BlockSpec((_ESM, _P), lambda i: (0, 0)),
            pl.BlockSpec((1, _P), lambda i: (0, 0)),
            pl.BlockSpec((1, _P), lambda i: (0, 0)),
            pl.BlockSpec((1, _P), lambda i: (0, 0)),
            pl.BlockSpec((_H, _P), lambda i: (0, 0)),
            pl.BlockSpec((_H, 1), lambda i: (0, 0)),
            pl.BlockSpec((1, _H), lambda i: (0, 0)),
            pl.BlockSpec((1, 1), lambda i: (0, 0)),
        ],
        out_specs=pl.BlockSpec((_B, 2 * _P), lambda i: (0, 0)),
        out_shape=jax.ShapeDtypeStruct((_B, 2 * _P), jnp.float32),
        scratch_shapes=[
            pltpu.VMEM((_B, 1), jnp.float32),
            pltpu.VMEM((_B, 1), jnp.float32),
            pltpu.VMEM((_B, 1), jnp.float32),
            pltpu.VMEM((_B, _P), jnp.float32),
            pltpu.VMEM((_B, _P), jnp.float32),
        ],
        compiler_params=pltpu.CompilerParams(
            dimension_semantics=("arbitrary",)),
    )(x_esm, ids, W1, b1.reshape(1, _P), gamma.reshape(1, _P),
      beta.reshape(1, _P), Ws1.T, bs1.reshape(_H, 1), Ws2.T,
      bs2.reshape(1, 1))
    return out
